# baseline (speedup 1.0000x reference)
import jax
import jax.numpy as jnp
from jax.experimental import pallas as pl
from jax.experimental.pallas import tpu as pltpu

_SCALE = 80.0
_R2 = 40.0 * 40.0
_BIG = 1e10
_TN = 512
_MW = 512


def _chamfer_kernel(aac_ref, aar_ref, bb_ref, pr_ref, g2_ref, out_ref,
                    cminU_s, cminM_s, accM_s, sums_s):
    N = aac_ref.shape[1]
    M = bb_ref.shape[2]
    j = pl.program_id(1)
    nj = pl.num_programs(1)
    row0 = j * _TN

    bb = bb_ref[0]

    @pl.when(j == 0)
    def _init():
        cminU_s[:, :] = jnp.full((1, M), _BIG, jnp.float32)
        cminM_s[:, :] = jnp.full((1, M), _BIG, jnp.float32)
        sums_s[0, 0] = 0.0
        sums_s[1, 0] = 0.0
        aar = aar_ref[0]
        sums_s[2, 0] = jnp.sum((aar < _R2).astype(jnp.float32))
        sums_s[3, 0] = jnp.sum((bb < _R2).astype(jnp.float32))

    kp = sums_s[2, 0]
    kg = sums_s[3, 0]
    aa_t = aac_ref[0, pl.ds(row0, _TN), :]
    p_r = pr_ref[0, pl.ds(row0, _TN), :]

    accM_s[:, :] = jnp.full((_TN, 128), _BIG, jnp.float32)

    tile_all_valid = jnp.float32(row0) + _TN <= kp
    tile_boundary = (jnp.float32(row0) < kp) & (kp < jnp.float32(row0) + _TN)
    ridx = (jax.lax.broadcasted_iota(jnp.int32, (_TN, 1), 0) + row0
            ).astype(jnp.float32)
    pen_r = jnp.where(ridx < kp, 0.0, _BIG)

    def fold128(x):
        m = x[:, 0:128]
        for s in range(1, _MW // 128):
            m = jnp.minimum(m, x[:, s * 128:(s + 1) * 128])
        return m

    def chunk(c, accU):
        c0 = c * _MW
        ab2 = jax.lax.dot_general(
            p_r, g2_ref[0, :, pl.ds(c0, _MW)], (((1,), (0,)), ((), ())),
            preferred_element_type=jnp.float32)
        bb_c = bb_ref[0, :, pl.ds(c0, _MW)]

        ec = bb_c + ab2
        m128 = fold128(ec)
        accU = jnp.minimum(accU, m128)

        @pl.when(jnp.float32(c0) + _MW <= kg)
        def _valid_chunk():
            accM_s[:, :] = jnp.minimum(accM_s[:, :], m128)

        @pl.when((jnp.float32(c0) < kg) & (kg < jnp.float32(c0) + _MW))
        def _boundary_chunk():
            lane = (jax.lax.broadcasted_iota(jnp.int32, (1, _MW), 1) + c0
                    ).astype(jnp.float32)
            ecp = ec + jnp.where(lane < kg, 0.0, _BIG)
            accM_s[:, :] = jnp.minimum(accM_s[:, :], fold128(ecp))

        fc = aa_t + ab2
        colmin = jnp.min(fc, axis=0, keepdims=True)
        cminU_s[0:1, pl.ds(c0, _MW)] = jnp.minimum(
            cminU_s[0:1, pl.ds(c0, _MW)], colmin)

        @pl.when(tile_all_valid)
        def _valid_tile():
            cminM_s[0:1, pl.ds(c0, _MW)] = jnp.minimum(
                cminM_s[0:1, pl.ds(c0, _MW)], colmin)

        @pl.when(tile_boundary)
        def _boundary_tile():
            colmin_p = jnp.min(fc + pen_r, axis=0, keepdims=True)
            cminM_s[0:1, pl.ds(c0, _MW)] = jnp.minimum(
                cminM_s[0:1, pl.ds(c0, _MW)], colmin_p)

        return accU

    accU = jax.lax.fori_loop(0, M // _MW, chunk,
                             jnp.full((_TN, 128), _BIG, jnp.float32))

    rminU = jnp.maximum(aa_t + jnp.min(accU, axis=1, keepdims=True), 0.0)
    rminM = jnp.maximum(
        aa_t + jnp.min(accM_s[:, :], axis=1, keepdims=True), 0.0)
    sums_s[0, 0] = sums_s[0, 0] + jnp.sum(rminU)
    sums_s[1, 0] = sums_s[1, 0] + jnp.sum(jnp.where(ridx < kp, rminM, 0.0))

    @pl.when(j == nj - 1)
    def _finalize():
        cidx = jax.lax.broadcasted_iota(jnp.int32, (1, M), 1
                                        ).astype(jnp.float32)
        mg = cidx < kg
        cminU = jnp.maximum(bb + cminU_s[:, :], 0.0)
        cminM = jnp.maximum(bb + cminM_s[:, :], 0.0)
        sum_c_u = jnp.sum(cminU)
        sum_c_m = jnp.sum(jnp.where(mg, cminM, 0.0))
        non_filtered = sums_s[0, 0] / N + sum_c_u / M
        filtered = (sums_s[1, 0] / jnp.maximum(sums_s[2, 0], 1.0)
                    + sum_c_m / jnp.maximum(sums_s[3, 0], 1.0))
        loss = 0.7 * filtered + 0.3 * non_filtered
        out_ref[:, :, :] = jnp.broadcast_to(loss, (1, 1, 1))


def kernel(image_pred, image_gt):
    B, N, _ = image_pred.shape
    M = image_gt.shape[1]

    p = image_pred * _SCALE
    g = image_gt * _SCALE
    aa = jnp.sum(p * p, axis=-1)
    bb = jnp.sum(g * g, axis=-1)

    perm_p = jnp.argsort((aa >= _R2).astype(jnp.int32), axis=1)
    perm_g = jnp.argsort((bb >= _R2).astype(jnp.int32), axis=1)
    p_s = jnp.take_along_axis(p, perm_p[:, :, None], axis=1)
    g_s = jnp.take_along_axis(g, perm_g[:, :, None], axis=1)
    aa_s = jnp.take_along_axis(aa, perm_p, axis=1)
    bb_s = jnp.take_along_axis(bb, perm_g, axis=1)

    p_r = jnp.pad(p_s.astype(jnp.bfloat16), ((0, 0), (0, 0), (0, 5)))
    g2 = (-2.0 * g_s.astype(jnp.bfloat16).astype(jnp.float32)
          ).astype(jnp.bfloat16)
    g2t = jnp.pad(jnp.swapaxes(g2, 1, 2), ((0, 0), (0, 5), (0, 0)))

    aac = aa_s[:, :, None]
    aar = aa_s[:, None, :]
    bbr = bb_s[:, None, :]

    per_batch = pl.pallas_call(
        _chamfer_kernel,
        grid=(B, N // _TN),
        in_specs=[
            pl.BlockSpec((1, N, 1), lambda b, j: (b, 0, 0)),
            pl.BlockSpec((1, 1, N), lambda b, j: (b, 0, 0)),
            pl.BlockSpec((1, 1, M), lambda b, j: (b, 0, 0)),
            pl.BlockSpec((1, N, 8), lambda b, j: (b, 0, 0)),
            pl.BlockSpec((1, 8, M), lambda b, j: (b, 0, 0)),
        ],
        out_specs=pl.BlockSpec((1, 1, 1), lambda b, j: (b, 0, 0)),
        out_shape=jax.ShapeDtypeStruct((B, 1, 1), jnp.float32),
        scratch_shapes=[
            pltpu.VMEM((1, M), jnp.float32),
            pltpu.VMEM((1, M), jnp.float32),
            pltpu.VMEM((_TN, 128), jnp.float32),
            pltpu.SMEM((8, 1), jnp.float32),
        ],
        compiler_params=pltpu.CompilerParams(
            dimension_semantics=("arbitrary", "arbitrary")),
    )(aac, aar, bbr, p_r, g2t)
    return jnp.mean(per_batch)


# speedup vs baseline: 4.4492x; 4.4492x over previous
import jax
import jax.numpy as jnp
from jax.experimental import pallas as pl
from jax.experimental.pallas import tpu as pltpu

_SCALE = 80.0
_R2 = 40.0 * 40.0
_BIG = 1e10
_TN = 2048


def _chamfer_kernel(p_ref, gt_ref, pr_ref, g2_ref, out_ref):
    N = p_ref.shape[1]
    M = gt_ref.shape[2]

    gx = gt_ref[0, 0:1, :] * _SCALE
    gy = gt_ref[0, 1:2, :] * _SCALE
    gz = gt_ref[0, 2:3, :] * _SCALE
    bb = gx * gx + gy * gy + gz * gz
    mg = bb < _R2
    bbm = jnp.where(mg, bb, bb + _BIG)
    g2 = g2_ref[0]

    def body(j, carry):
        cmin_u, cmin_m, rsum_u, rsum_m, cnt_p = carry
        p_blk = p_ref[0, pl.ds(j * _TN, _TN), :] * _SCALE
        px = p_blk[:, 0:1]
        py = p_blk[:, 1:2]
        pz = p_blk[:, 2:3]
        aa = px * px + py * py + pz * pz
        mp = aa < _R2
        aam = jnp.where(mp, aa, aa + _BIG)

        p_r = pr_ref[0, pl.ds(j * _TN, _TN), :]
        ab2 = jax.lax.dot_general(
            p_r, g2, (((1,), (0,)), ((), ())),
            preferred_element_type=jnp.float32)

        rmin_u = jnp.maximum(
            aa + jnp.min(bb + ab2, axis=1, keepdims=True), 0.0)
        rmin_m = jnp.maximum(
            aa + jnp.min(bbm + ab2, axis=1, keepdims=True), 0.0)
        cmin_u = jnp.minimum(cmin_u, jnp.min(aa + ab2, axis=0, keepdims=True))
        cmin_m = jnp.minimum(cmin_m, jnp.min(aam + ab2, axis=0, keepdims=True))

        rsum_u = rsum_u + jnp.sum(rmin_u)
        rsum_m = rsum_m + jnp.sum(jnp.where(mp, rmin_m, 0.0))
        cnt_p = cnt_p + jnp.sum(mp.astype(jnp.float32))
        return cmin_u, cmin_m, rsum_u, rsum_m, cnt_p

    init = (
        jnp.full((1, M), _BIG, jnp.float32),
        jnp.full((1, M), _BIG, jnp.float32),
        jnp.float32(0.0),
        jnp.float32(0.0),
        jnp.float32(0.0),
    )
    cmin_u, cmin_m, rsum_u, rsum_m, cnt_p = jax.lax.fori_loop(
        0, N // _TN, body, init)

    cmin_u = jnp.maximum(bb + cmin_u, 0.0)
    cmin_m = jnp.maximum(bb + cmin_m, 0.0)
    sum_c_u = jnp.sum(cmin_u)
    sum_c_m = jnp.sum(jnp.where(mg, cmin_m, 0.0))
    cnt_g = jnp.sum(mg.astype(jnp.float32))

    non_filtered = rsum_u / N + sum_c_u / M
    filtered = (rsum_m / jnp.maximum(cnt_p, 1.0)
                + sum_c_m / jnp.maximum(cnt_g, 1.0))
    loss = 0.7 * filtered + 0.3 * non_filtered
    out_ref[:, :, :] = jnp.broadcast_to(loss, (1, 1, 1))


def kernel(image_pred, image_gt):
    B, N, _ = image_pred.shape
    M = image_gt.shape[1]
    gt_t = jnp.swapaxes(image_gt, 1, 2)

    p_r = (image_pred * _SCALE).astype(jnp.bfloat16)
    g2 = (-2.0 * (image_gt * _SCALE).astype(jnp.bfloat16)
          .astype(jnp.float32)).astype(jnp.bfloat16)
    p_r = jnp.pad(p_r, ((0, 0), (0, 0), (0, 5)))
    g2t = jnp.pad(jnp.swapaxes(g2, 1, 2),
                  ((0, 0), (0, 5), (0, 0)))

    per_batch = pl.pallas_call(
        _chamfer_kernel,
        grid=(B,),
        in_specs=[
            pl.BlockSpec((1, N, 3), lambda b: (b, 0, 0)),
            pl.BlockSpec((1, 3, M), lambda b: (b, 0, 0)),
            pl.BlockSpec((1, N, 8), lambda b: (b, 0, 0)),
            pl.BlockSpec((1, 8, M), lambda b: (b, 0, 0)),
        ],
        out_specs=pl.BlockSpec((1, 1, 1), lambda b: (b, 0, 0)),
        out_shape=jax.ShapeDtypeStruct((B, 1, 1), jnp.float32),
        compiler_params=pltpu.CompilerParams(
            dimension_semantics=("parallel",)),
    )(image_pred, gt_t, p_r, g2t)
    return jnp.mean(per_batch)


# speedup vs baseline: 4.7818x; 1.0748x over previous
import jax
import jax.numpy as jnp
from jax.experimental import pallas as pl
from jax.experimental.pallas import tpu as pltpu

_SCALE = 80.0
_R2 = 40.0 * 40.0
_BIG = 1e10
_TN = 2048


def _chamfer_kernel(p_ref, gt_ref, out_ref):
    N = p_ref.shape[1]
    M = gt_ref.shape[2]

    gx = gt_ref[0, 0:1, :] * _SCALE
    gy = gt_ref[0, 1:2, :] * _SCALE
    gz = gt_ref[0, 2:3, :] * _SCALE
    bb = gx * gx + gy * gy + gz * gz
    mg = bb < _R2
    bbm = jnp.where(mg, bb, bb + _BIG)

    def neg2bf16(v):
        return (-2.0 * v.astype(jnp.bfloat16).astype(jnp.float32)
                ).astype(jnp.bfloat16)

    g2 = jnp.concatenate(
        [neg2bf16(gx), neg2bf16(gy), neg2bf16(gz)], axis=0)

    def body(j, carry):
        cmin_u, cmin_m, rsum_u, rsum_m, cnt_p = carry
        p_blk = p_ref[0, pl.ds(j * _TN, _TN), :] * _SCALE
        px = p_blk[:, 0:1]
        py = p_blk[:, 1:2]
        pz = p_blk[:, 2:3]
        aa = px * px + py * py + pz * pz
        mp = aa < _R2
        aam = jnp.where(mp, aa, aa + _BIG)

        p_r = p_blk.astype(jnp.bfloat16)
        ab2 = jax.lax.dot_general(
            p_r, g2, (((1,), (0,)), ((), ())),
            preferred_element_type=jnp.float32)

        rmin_u = jnp.maximum(
            aa + jnp.min(bb + ab2, axis=1, keepdims=True), 0.0)
        rmin_m = jnp.maximum(
            aa + jnp.min(bbm + ab2, axis=1, keepdims=True), 0.0)
        cmin_u = jnp.minimum(cmin_u, jnp.min(aa + ab2, axis=0, keepdims=True))
        cmin_m = jnp.minimum(cmin_m, jnp.min(aam + ab2, axis=0, keepdims=True))

        rsum_u = rsum_u + jnp.sum(rmin_u)
        rsum_m = rsum_m + jnp.sum(jnp.where(mp, rmin_m, 0.0))
        cnt_p = cnt_p + jnp.sum(mp.astype(jnp.float32))
        return cmin_u, cmin_m, rsum_u, rsum_m, cnt_p

    init = (
        jnp.full((1, M), _BIG, jnp.float32),
        jnp.full((1, M), _BIG, jnp.float32),
        jnp.float32(0.0),
        jnp.float32(0.0),
        jnp.float32(0.0),
    )
    cmin_u, cmin_m, rsum_u, rsum_m, cnt_p = jax.lax.fori_loop(
        0, N // _TN, body, init)

    cmin_u = jnp.maximum(bb + cmin_u, 0.0)
    cmin_m = jnp.maximum(bb + cmin_m, 0.0)
    sum_c_u = jnp.sum(cmin_u)
    sum_c_m = jnp.sum(jnp.where(mg, cmin_m, 0.0))
    cnt_g = jnp.sum(mg.astype(jnp.float32))

    non_filtered = rsum_u / N + sum_c_u / M
    filtered = (rsum_m / jnp.maximum(cnt_p, 1.0)
                + sum_c_m / jnp.maximum(cnt_g, 1.0))
    loss = 0.7 * filtered + 0.3 * non_filtered
    out_ref[:, :, :] = jnp.broadcast_to(loss, (1, 1, 1))


def kernel(image_pred, image_gt):
    B, N, _ = image_pred.shape
    M = image_gt.shape[1]
    gt_t = jnp.swapaxes(image_gt, 1, 2)

    per_batch = pl.pallas_call(
        _chamfer_kernel,
        grid=(B,),
        in_specs=[
            pl.BlockSpec((1, N, 3), lambda b: (b, 0, 0)),
            pl.BlockSpec((1, 3, M), lambda b: (b, 0, 0)),
        ],
        out_specs=pl.BlockSpec((1, 1, 1), lambda b: (b, 0, 0)),
        out_shape=jax.ShapeDtypeStruct((B, 1, 1), jnp.float32),
        compiler_params=pltpu.CompilerParams(
            dimension_semantics=("parallel",)),
    )(image_pred, gt_t)
    return jnp.mean(per_batch)


# speedup vs baseline: 5.0132x; 1.0484x over previous
import jax
import jax.numpy as jnp
from jax.experimental import pallas as pl
from jax.experimental.pallas import tpu as pltpu

_SCALE = 80.0
_R2 = 40.0 * 40.0
_BIG = 1e10
_TN = 2048


def _chamfer_kernel(p_ref, gt_ref, out_ref):
    N = p_ref.shape[1]
    M = gt_ref.shape[2]

    gx = gt_ref[0, 0:1, :] * _SCALE
    gy = gt_ref[0, 1:2, :] * _SCALE
    gz = gt_ref[0, 2:3, :] * _SCALE
    bb = gx * gx + gy * gy + gz * gz
    mg = bb < _R2
    bbm = jnp.where(mg, bb, bb + _BIG)

    def neg2bf16(v):
        return (-2.0 * v.astype(jnp.bfloat16).astype(jnp.float32)
                ).astype(jnp.bfloat16)

    g2 = jnp.concatenate(
        [neg2bf16(gx), neg2bf16(gy), neg2bf16(gz)], axis=0)

    def body(j, carry):
        cmin_u, cmin_m, rsum_u, rsum_m, cnt_p = carry
        p_blk = p_ref[0, pl.ds(j * _TN, _TN), :] * _SCALE
        px = p_blk[:, 0:1]
        py = p_blk[:, 1:2]
        pz = p_blk[:, 2:3]
        aa = px * px + py * py + pz * pz
        mp = aa < _R2
        aam = jnp.where(mp, aa, aa + _BIG)

        p_r = p_blk.astype(jnp.bfloat16)
        ab2 = jax.lax.dot_general(
            p_r, g2, (((1,), (0,)), ((), ())),
            preferred_element_type=jnp.float32)

        rmin_u = jnp.maximum(
            aa + jnp.min(bb + ab2, axis=1, keepdims=True), 0.0)
        rmin_m = jnp.maximum(
            aa + jnp.min(bbm + ab2, axis=1, keepdims=True), 0.0)
        cmin_u = jnp.minimum(cmin_u, jnp.min(aa + ab2, axis=0, keepdims=True))
        cmin_m = jnp.minimum(cmin_m, jnp.min(aam + ab2, axis=0, keepdims=True))

        rsum_u = rsum_u + jnp.sum(rmin_u)
        rsum_m = rsum_m + jnp.sum(jnp.where(mp, rmin_m, 0.0))
        cnt_p = cnt_p + jnp.sum(mp.astype(jnp.float32))
        return cmin_u, cmin_m, rsum_u, rsum_m, cnt_p

    init = (
        jnp.full((1, M), _BIG, jnp.float32),
        jnp.full((1, M), _BIG, jnp.float32),
        jnp.float32(0.0),
        jnp.float32(0.0),
        jnp.float32(0.0),
    )
    cmin_u, cmin_m, rsum_u, rsum_m, cnt_p = jax.lax.fori_loop(
        0, N // _TN, body, init)

    cmin_u = jnp.maximum(bb + cmin_u, 0.0)
    cmin_m = jnp.maximum(bb + cmin_m, 0.0)
    sum_c_u = jnp.sum(cmin_u)
    sum_c_m = jnp.sum(jnp.where(mg, cmin_m, 0.0))
    cnt_g = jnp.sum(mg.astype(jnp.float32))

    non_filtered = rsum_u / N + sum_c_u / M
    filtered = (rsum_m / jnp.maximum(cnt_p, 1.0)
                + sum_c_m / jnp.maximum(cnt_g, 1.0))
    loss = (0.7 * filtered + 0.3 * non_filtered) / pl.num_programs(0)

    @pl.when(pl.program_id(0) == 0)
    def _first():
        out_ref[:, :, :] = jnp.broadcast_to(loss, (1, 1, 1))

    @pl.when(pl.program_id(0) != 0)
    def _rest():
        out_ref[:, :, :] = out_ref[:, :, :] + loss


def kernel(image_pred, image_gt):
    B, N, _ = image_pred.shape
    M = image_gt.shape[1]
    gt_t = jnp.swapaxes(image_gt, 1, 2)

    per_batch = pl.pallas_call(
        _chamfer_kernel,
        grid=(B,),
        in_specs=[
            pl.BlockSpec((1, N, 3), lambda b: (b, 0, 0)),
            pl.BlockSpec((1, 3, M), lambda b: (b, 0, 0)),
        ],
        out_specs=pl.BlockSpec((1, 1, 1), lambda b: (0, 0, 0)),
        out_shape=jax.ShapeDtypeStruct((1, 1, 1), jnp.float32),
        compiler_params=pltpu.CompilerParams(
            dimension_semantics=("arbitrary",)),
    )(image_pred, gt_t)
    return per_batch.reshape(())
